# baseline (speedup 1.0000x reference)
import functools

import jax
import jax.numpy as jnp
from jax import lax
from jax.experimental import pallas as pl
from jax.experimental.pallas import tpu as pltpu
from jax.experimental.pallas import tpu_sc as plsc

B = 16384
F_CAT = 26
F_NUM = 13
V = 100000
D = 8
H = 128

NC, NS = 2, 16
NW = NC * NS
N_LOOK = B * F_CAT
N_PER_W = N_LOOK // NW
LANES = 16

_MESH = plsc.VectorSubcoreMesh(
    core_axis_name="c", subcore_axis_name="s",
    num_cores=NC, num_subcores=NS)
_SC_LIN = pltpu.CompilerParams(
    use_tc_tiling_on_sc=False, needs_layout_passes=False)
_SC_TILED = pltpu.CompilerParams(
    use_tc_tiling_on_sc=True, needs_layout_passes=False)


VT_FULL = V // 128
V_TAIL = V - VT_FULL * 128
N_UNIT = F_CAT * VT_FULL
KMAX = 2 * ((N_UNIT + 2 * NW - 1) // (2 * NW))


def _transpose_body(t2_hbm, out_hbm, tin0, tin1, tout0, tout1, tint, toutt,
                    r0, r1, w0, w1):
    wid = lax.axis_index("s") * NC + lax.axis_index("c")
    tins, touts, rsems, wsems = (tin0, tin1), (tout0, tout1), (r0, r1), (w0, w1)

    def unit_uv(k):
        u = jnp.minimum(wid + NW * k, N_UNIT - 1)
        return lax.div(u, VT_FULL), lax.rem(u, VT_FULL)

    def start_read(k, cur):
        f, tv = unit_uv(k)
        pltpu.async_copy(
            t2_hbm.at[pl.ds(f * D, D), pl.ds(tv * 128, 128)],
            tins[cur], rsems[cur])

    def reorder(tin, tout, nvecs):
        lanes = lax.iota(jnp.int32, LANES)
        ri = lax.rem(lanes, D)
        half = lax.div(lanes, D)
        for k2 in range(nvecs):
            tout[pl.ds(k2 * LANES, LANES)] = plsc.load_gather(
                tin, [ri, 2 * k2 + half])

    start_read(0, 0)
    start_read(1, 1)

    def pipe(j, carry):
        for cur in (0, 1):
            k = 2 * j + cur
            pltpu.make_async_copy(
                t2_hbm.at[pl.ds(0, D), pl.ds(0, 128)],
                tins[cur], rsems[cur]).wait()

            @pl.when(j > 0)
            def _():
                pltpu.make_async_copy(
                    touts[cur], out_hbm.at[pl.ds(0, 1024)],
                    wsems[cur]).wait()

            reorder(tins[cur], touts[cur], 64)
            f, tv = unit_uv(k)
            pltpu.async_copy(
                touts[cur],
                out_hbm.at[pl.ds(f * (V * D) + tv * 1024, 1024)],
                wsems[cur])
            start_read(k + 2, cur)
        return carry

    lax.fori_loop(0, KMAX // 2, pipe, 0)
    for cur in (0, 1):
        pltpu.make_async_copy(
            t2_hbm.at[pl.ds(0, D), pl.ds(0, 128)], tins[cur],
            rsems[cur]).wait()
        pltpu.make_async_copy(
            touts[cur], out_hbm.at[pl.ds(0, 1024)], wsems[cur]).wait()

    @pl.when(wid < F_CAT)
    def _():
        f = wid
        pltpu.sync_copy(
            t2_hbm.at[pl.ds(f * D, D), pl.ds(VT_FULL * 128, V_TAIL)], tint)
        reorder(tint, toutt, (V_TAIL * D) // LANES)
        pltpu.sync_copy(
            toutt, out_hbm.at[pl.ds(f * (V * D) + VT_FULL * 1024,
                                    V_TAIL * D)])


def _sc_transpose(t2):
    return pl.kernel(
        _transpose_body,
        out_type=jax.ShapeDtypeStruct((F_CAT * V * D,), jnp.float32),
        mesh=_MESH,
        scratch_types=[
            pltpu.VMEM((D, 128), jnp.float32),
            pltpu.VMEM((D, 128), jnp.float32),
            pltpu.VMEM((1024,), jnp.float32),
            pltpu.VMEM((1024,), jnp.float32),
            pltpu.VMEM((D, V_TAIL), jnp.float32),
            pltpu.VMEM((V_TAIL * D,), jnp.float32),
            pltpu.SemaphoreType.DMA,
            pltpu.SemaphoreType.DMA,
            pltpu.SemaphoreType.DMA,
            pltpu.SemaphoreType.DMA,
        ],
        compiler_params=_SC_TILED,
    )(t2)


def _fv_body(idx_hbm, fv_hbm, fv_v):
    wid = lax.axis_index("s") * NC + lax.axis_index("c")
    base = wid * N_PER_W
    pltpu.sync_copy(idx_hbm.at[pl.ds(base, N_PER_W)], fv_v)

    def flatfv(j, carry):
        sl = pl.ds(j * LANES, LANES)
        pos = base + j * LANES + lax.iota(jnp.int32, LANES)
        fv_v[sl] = fv_v[sl] + lax.rem(pos, F_CAT) * V
        return carry

    lax.fori_loop(0, N_PER_W // LANES, flatfv, 0)
    pltpu.sync_copy(fv_v, fv_hbm.at[pl.ds(base, N_PER_W)])


def _sc_fv(idx_flat):
    return pl.kernel(
        _fv_body,
        out_type=jax.ShapeDtypeStruct((N_LOOK,), jnp.int32),
        mesh=_MESH,
        scratch_types=[pltpu.VMEM((N_PER_W,), jnp.int32)],
        compiler_params=_SC_LIN,
    )(idx_flat)


N_CHUNK = 8
C_ROWS = N_PER_W // N_CHUNK


def _rowgather_body(fv_hbm, table_hbm, out_hbm, ridx_v, rows_v,
                    gsem, w0, w1):
    wid = lax.axis_index("s") * NC + lax.axis_index("c")
    base = wid * N_PER_W
    wsems = (w0, w1)
    wr = [None, None]
    pltpu.sync_copy(fv_hbm.at[pl.ds(base, C_ROWS)], ridx_v.at[0])
    for c in range(N_CHUNK):
        cur = c & 1
        if wr[cur] is not None:
            wr[cur].wait()
        g = pltpu.async_copy(
            table_hbm.at[ridx_v.at[cur]], rows_v.at[cur], gsem)
        if c + 1 < N_CHUNK:
            pltpu.sync_copy(
                fv_hbm.at[pl.ds(base + (c + 1) * C_ROWS, C_ROWS)],
                ridx_v.at[1 - cur])
        g.wait()
        wr[cur] = pltpu.async_copy(
            rows_v.at[cur],
            out_hbm.at[pl.ds(base + c * C_ROWS, C_ROWS)], wsems[cur])
    wr[0].wait()
    wr[1].wait()


def _sc_rowgather(fv, table2):
    return pl.kernel(
        _rowgather_body,
        out_type=jax.ShapeDtypeStruct((N_LOOK, D), jnp.float32),
        mesh=_MESH,
        scratch_types=[
            pltpu.VMEM((2, C_ROWS), jnp.int32),
            pltpu.VMEM((2, C_ROWS, D), jnp.float32),
            pltpu.SemaphoreType.DMA,
            pltpu.SemaphoreType.DMA,
            pltpu.SemaphoreType.DMA,
        ],
        compiler_params=_SC_LIN,
    )(fv, table2)


BLK = 1024


def _mlp_body(cat_ref, nv_ref, nw_ref, nb_ref, w1_ref, b1_ref, w2_ref,
              b2_ref, out_ref):
    catf = cat_ref[...]
    nv = nv_ref[...]
    fi = lax.broadcasted_iota(jnp.int32, (F_NUM, F_NUM * D), 0)
    ji = lax.broadcasted_iota(jnp.int32, (F_NUM, F_NUM * D), 1)
    expand = jnp.where(ji // D == fi, 1.0, 0.0)
    rep = jnp.dot(nv, expand, preferred_element_type=jnp.float32)
    numf = rep * nw_ref[...] + nb_ref[...]
    h = (jnp.dot(catf, w1_ref[0:F_CAT * D, :],
                 preferred_element_type=jnp.float32)
         + jnp.dot(numf, w1_ref[F_CAT * D:, :],
                   preferred_element_type=jnp.float32)
         + b1_ref[...])
    h = jnp.maximum(h, 0.0)
    o = jnp.dot(h, w2_ref[...], preferred_element_type=jnp.float32) + b2_ref[...]
    out_ref[...] = jax.nn.sigmoid(o)


def _mlp(cat_flat, num_values, nw, nb, W1, b1r, W2, b2r):
    grid = (B // BLK,)
    return pl.pallas_call(
        _mlp_body,
        grid=grid,
        in_specs=[
            pl.BlockSpec((BLK, F_CAT * D), lambda i: (i, 0)),
            pl.BlockSpec((BLK, F_NUM), lambda i: (i, 0)),
            pl.BlockSpec((1, F_NUM * D), lambda i: (0, 0)),
            pl.BlockSpec((1, F_NUM * D), lambda i: (0, 0)),
            pl.BlockSpec(((F_CAT + F_NUM) * D, H), lambda i: (0, 0)),
            pl.BlockSpec((1, H), lambda i: (0, 0)),
            pl.BlockSpec((H, 1), lambda i: (0, 0)),
            pl.BlockSpec((1, 1), lambda i: (0, 0)),
        ],
        out_specs=pl.BlockSpec((BLK, 1), lambda i: (i, 0)),
        out_shape=jax.ShapeDtypeStruct((B, 1), jnp.float32),
    )(cat_flat, num_values, nw, nb, W1, b1r, W2, b2r)


def kernel(cat_indices, num_values, emb_tables, num_W, num_b, W1, b1, W2, b2):
    idx_flat = cat_indices.reshape(N_LOOK).astype(jnp.int32)
    t2 = emb_tables.transpose(0, 2, 1).reshape(F_CAT * D, V)
    tv_lin = _sc_transpose(t2)
    fv = _sc_fv(idx_flat)
    table2 = tv_lin.reshape(F_CAT * V, D)
    cat_flat = _sc_rowgather(fv, table2).reshape(B, F_CAT * D)
    nw = num_W.reshape(1, F_NUM * D)
    nb = num_b.reshape(1, F_NUM * D)
    return _mlp(cat_flat, num_values, nw, nb, W1,
                b1.reshape(1, H), W2, b2.reshape(1, 1))


# speedup vs baseline: 1.0982x; 1.0982x over previous
import functools

import jax
import jax.numpy as jnp
from jax import lax
from jax.experimental import pallas as pl
from jax.experimental.pallas import tpu as pltpu
from jax.experimental.pallas import tpu_sc as plsc

B = 16384
F_CAT = 26
F_NUM = 13
V = 100000
D = 8
H = 128

NC, NS = 2, 16
NW = NC * NS
N_LOOK = B * F_CAT
N_PER_W = N_LOOK // NW
LANES = 16

_MESH = plsc.VectorSubcoreMesh(
    core_axis_name="c", subcore_axis_name="s",
    num_cores=NC, num_subcores=NS)
_SC_LIN = pltpu.CompilerParams(
    use_tc_tiling_on_sc=False, needs_layout_passes=False)
_SC_TILED = pltpu.CompilerParams(
    use_tc_tiling_on_sc=True, needs_layout_passes=False)


VT_FULL = V // 128
V_TAIL = V - VT_FULL * 128
TPU_T = 11
U_PER_F = VT_FULL // TPU_T
U_COLS = TPU_T * 128
U_WORDS = U_COLS * D
N_UNIT = F_CAT * U_PER_F
KMAX = 2 * ((N_UNIT + 2 * NW - 1) // (2 * NW))


def _transpose_body(t2_hbm, out_hbm, tin0, tin1, tout0, tout1, tint, toutt,
                    r0, r1, w0, w1):
    wid = lax.axis_index("s") * NC + lax.axis_index("c")
    tins, touts, rsems, wsems = (tin0, tin1), (tout0, tout1), (r0, r1), (w0, w1)

    def unit_uv(k):
        u = jnp.minimum(wid + NW * k, N_UNIT - 1)
        return lax.div(u, U_PER_F), lax.rem(u, U_PER_F)

    def start_read(k, cur):
        f, tq = unit_uv(k)
        pltpu.async_copy(
            t2_hbm.at[pl.ds(f * D, D), pl.ds(tq * U_COLS, U_COLS)],
            tins[cur], rsems[cur])

    def reorder_tile(tin, tout, col0, out0):
        lanes = lax.iota(jnp.int32, LANES)
        ri = lax.rem(lanes, D)
        half = lax.div(lanes, D)
        for k2 in range(64):
            tout[pl.ds(out0 + k2 * LANES, LANES)] = plsc.load_gather(
                tin, [ri, col0 + 2 * k2 + half])

    def reorder(tin, tout):
        def rbody(j2, carry):
            reorder_tile(tin, tout, j2 * 128, j2 * 1024)
            return carry
        lax.fori_loop(0, TPU_T, rbody, 0)

    start_read(0, 0)
    start_read(1, 1)

    def pipe(j, carry):
        for cur in (0, 1):
            k = 2 * j + cur
            pltpu.make_async_copy(
                t2_hbm.at[pl.ds(0, D), pl.ds(0, U_COLS)],
                tins[cur], rsems[cur]).wait()

            @pl.when(j > 0)
            def _():
                pltpu.make_async_copy(
                    touts[cur], out_hbm.at[pl.ds(0, U_WORDS)],
                    wsems[cur]).wait()

            reorder(tins[cur], touts[cur])
            f, tq = unit_uv(k)
            pltpu.async_copy(
                touts[cur],
                out_hbm.at[pl.ds(f * (V * D) + tq * U_WORDS, U_WORDS)],
                wsems[cur])
            start_read(k + 2, cur)
        return carry

    lax.fori_loop(0, KMAX // 2, pipe, 0)
    for cur in (0, 1):
        pltpu.make_async_copy(
            t2_hbm.at[pl.ds(0, D), pl.ds(0, U_COLS)], tins[cur],
            rsems[cur]).wait()
        pltpu.make_async_copy(
            touts[cur], out_hbm.at[pl.ds(0, U_WORDS)], wsems[cur]).wait()

    @pl.when(wid < F_CAT)
    def _():
        f = wid
        pltpu.sync_copy(
            t2_hbm.at[pl.ds(f * D, D), pl.ds(VT_FULL * 128, V_TAIL)], tint)
        lanes = lax.iota(jnp.int32, LANES)
        ri = lax.rem(lanes, D)
        half = lax.div(lanes, D)
        for k2 in range((V_TAIL * D) // LANES):
            toutt[pl.ds(k2 * LANES, LANES)] = plsc.load_gather(
                tint, [ri, 2 * k2 + half])
        pltpu.sync_copy(
            toutt, out_hbm.at[pl.ds(f * (V * D) + VT_FULL * 1024,
                                    V_TAIL * D)])


def _sc_transpose(t2):
    return pl.kernel(
        _transpose_body,
        out_type=jax.ShapeDtypeStruct((F_CAT * V * D,), jnp.float32),
        mesh=_MESH,
        scratch_types=[
            pltpu.VMEM((D, U_COLS), jnp.float32),
            pltpu.VMEM((D, U_COLS), jnp.float32),
            pltpu.VMEM((U_WORDS,), jnp.float32),
            pltpu.VMEM((U_WORDS,), jnp.float32),
            pltpu.VMEM((D, V_TAIL), jnp.float32),
            pltpu.VMEM((V_TAIL * D,), jnp.float32),
            pltpu.SemaphoreType.DMA,
            pltpu.SemaphoreType.DMA,
            pltpu.SemaphoreType.DMA,
            pltpu.SemaphoreType.DMA,
        ],
        compiler_params=_SC_TILED,
    )(t2)


def _fv_body(idx_hbm, fv_hbm, fv_v):
    wid = lax.axis_index("s") * NC + lax.axis_index("c")
    base = wid * N_PER_W
    pltpu.sync_copy(idx_hbm.at[pl.ds(base, N_PER_W)], fv_v)

    def flatfv(j, carry):
        sl = pl.ds(j * LANES, LANES)
        pos = base + j * LANES + lax.iota(jnp.int32, LANES)
        fv_v[sl] = fv_v[sl] + lax.rem(pos, F_CAT) * V
        return carry

    lax.fori_loop(0, N_PER_W // LANES, flatfv, 0)
    pltpu.sync_copy(fv_v, fv_hbm.at[pl.ds(base, N_PER_W)])


def _sc_fv(idx_flat):
    return pl.kernel(
        _fv_body,
        out_type=jax.ShapeDtypeStruct((N_LOOK,), jnp.int32),
        mesh=_MESH,
        scratch_types=[pltpu.VMEM((N_PER_W,), jnp.int32)],
        compiler_params=_SC_LIN,
    )(idx_flat)


N_CHUNK = 8
C_ROWS = N_PER_W // N_CHUNK


def _rowgather_body(fv_hbm, table_hbm, out_hbm, ridx_v, rows_v,
                    gsem, w0, w1):
    wid = lax.axis_index("s") * NC + lax.axis_index("c")
    base = wid * N_PER_W
    wsems = (w0, w1)
    wr = [None, None]
    pltpu.sync_copy(fv_hbm.at[pl.ds(base, C_ROWS)], ridx_v.at[0])
    for c in range(N_CHUNK):
        cur = c & 1
        if wr[cur] is not None:
            wr[cur].wait()
        g = pltpu.async_copy(
            table_hbm.at[ridx_v.at[cur]], rows_v.at[cur], gsem)
        if c + 1 < N_CHUNK:
            pltpu.sync_copy(
                fv_hbm.at[pl.ds(base + (c + 1) * C_ROWS, C_ROWS)],
                ridx_v.at[1 - cur])
        g.wait()
        wr[cur] = pltpu.async_copy(
            rows_v.at[cur],
            out_hbm.at[pl.ds(base + c * C_ROWS, C_ROWS)], wsems[cur])
    wr[0].wait()
    wr[1].wait()


def _sc_rowgather(fv, table2):
    return pl.kernel(
        _rowgather_body,
        out_type=jax.ShapeDtypeStruct((N_LOOK, D), jnp.float32),
        mesh=_MESH,
        scratch_types=[
            pltpu.VMEM((2, C_ROWS), jnp.int32),
            pltpu.VMEM((2, C_ROWS, D), jnp.float32),
            pltpu.SemaphoreType.DMA,
            pltpu.SemaphoreType.DMA,
            pltpu.SemaphoreType.DMA,
        ],
        compiler_params=_SC_LIN,
    )(fv, table2)


BLK = 1024


def _mlp_body(cat_ref, nv_ref, nw_ref, nb_ref, w1_ref, b1_ref, w2_ref,
              b2_ref, out_ref):
    catf = cat_ref[...]
    nv = nv_ref[...]
    fi = lax.broadcasted_iota(jnp.int32, (F_NUM, F_NUM * D), 0)
    ji = lax.broadcasted_iota(jnp.int32, (F_NUM, F_NUM * D), 1)
    expand = jnp.where(ji // D == fi, 1.0, 0.0)
    rep = jnp.dot(nv, expand, preferred_element_type=jnp.float32)
    numf = rep * nw_ref[...] + nb_ref[...]
    h = (jnp.dot(catf, w1_ref[0:F_CAT * D, :],
                 preferred_element_type=jnp.float32)
         + jnp.dot(numf, w1_ref[F_CAT * D:, :],
                   preferred_element_type=jnp.float32)
         + b1_ref[...])
    h = jnp.maximum(h, 0.0)
    o = jnp.dot(h, w2_ref[...], preferred_element_type=jnp.float32) + b2_ref[...]
    out_ref[...] = jax.nn.sigmoid(o)


def _mlp(cat_flat, num_values, nw, nb, W1, b1r, W2, b2r):
    grid = (B // BLK,)
    return pl.pallas_call(
        _mlp_body,
        grid=grid,
        in_specs=[
            pl.BlockSpec((BLK, F_CAT * D), lambda i: (i, 0)),
            pl.BlockSpec((BLK, F_NUM), lambda i: (i, 0)),
            pl.BlockSpec((1, F_NUM * D), lambda i: (0, 0)),
            pl.BlockSpec((1, F_NUM * D), lambda i: (0, 0)),
            pl.BlockSpec(((F_CAT + F_NUM) * D, H), lambda i: (0, 0)),
            pl.BlockSpec((1, H), lambda i: (0, 0)),
            pl.BlockSpec((H, 1), lambda i: (0, 0)),
            pl.BlockSpec((1, 1), lambda i: (0, 0)),
        ],
        out_specs=pl.BlockSpec((BLK, 1), lambda i: (i, 0)),
        out_shape=jax.ShapeDtypeStruct((B, 1), jnp.float32),
    )(cat_flat, num_values, nw, nb, W1, b1r, W2, b2r)


def kernel(cat_indices, num_values, emb_tables, num_W, num_b, W1, b1, W2, b2):
    idx_flat = cat_indices.reshape(N_LOOK).astype(jnp.int32)
    t2 = emb_tables.transpose(0, 2, 1).reshape(F_CAT * D, V)
    tv_lin = _sc_transpose(t2)
    fv = _sc_fv(idx_flat)
    table2 = tv_lin.reshape(F_CAT * V, D)
    cat_flat = _sc_rowgather(fv, table2).reshape(B, F_CAT * D)
    nw = num_W.reshape(1, F_NUM * D)
    nb = num_b.reshape(1, F_NUM * D)
    return _mlp(cat_flat, num_values, nw, nb, W1,
                b1.reshape(1, H), W2, b2.reshape(1, 1))


# speedup vs baseline: 1.5833x; 1.4417x over previous
import functools

import jax
import jax.numpy as jnp
from jax import lax
from jax.experimental import pallas as pl
from jax.experimental.pallas import tpu as pltpu
from jax.experimental.pallas import tpu_sc as plsc

B = 16384
F_CAT = 26
F_NUM = 13
V = 100000
D = 8
H = 128

NC, NS = 2, 16
NW = NC * NS
N_LOOK = B * F_CAT
N_PER_W = N_LOOK // NW
LANES = 16

_MESH = plsc.VectorSubcoreMesh(
    core_axis_name="c", subcore_axis_name="s",
    num_cores=NC, num_subcores=NS)
_SC_LIN = pltpu.CompilerParams(
    use_tc_tiling_on_sc=False, needs_layout_passes=False)
_SC_TILED = pltpu.CompilerParams(
    use_tc_tiling_on_sc=True, needs_layout_passes=False)


VT_FULL = V // 128
V_TAIL = V - VT_FULL * 128
TPU_T = 11
U_PER_F = VT_FULL // TPU_T
U_COLS = TPU_T * 128
U_WORDS = U_COLS * D
N_UNIT = F_CAT * U_PER_F
KMAX = 2 * ((N_UNIT + 2 * NW - 1) // (2 * NW))


def _transpose_body(t2_hbm, out_hbm, tin0, tin1, tout0, tout1, tint, toutt,
                    r0, r1, w0, w1):
    wid = lax.axis_index("s") * NC + lax.axis_index("c")
    tins, touts, rsems, wsems = (tin0, tin1), (tout0, tout1), (r0, r1), (w0, w1)

    def unit_uv(k):
        u = jnp.minimum(wid + NW * k, N_UNIT - 1)
        return lax.div(u, U_PER_F), lax.rem(u, U_PER_F)

    def start_read(k, cur):
        f, tq = unit_uv(k)
        pltpu.async_copy(
            t2_hbm.at[pl.ds(f * D, D), pl.ds(tq * U_COLS, U_COLS)],
            tins[cur], rsems[cur])

    def reorder_tile(tin, tout, col0, out0):
        lanes8 = lax.iota(jnp.int32, LANES) * D
        for d in range(D):
            for k3 in range(128 // LANES):
                x = tin[d, pl.ds(col0 + k3 * LANES, LANES)]
                plsc.store_scatter(
                    tout, [out0 + (k3 * LANES) * D + d + lanes8], x)

    def reorder(tin, tout):
        def rbody(j2, carry):
            reorder_tile(tin, tout, j2 * 128, j2 * 1024)
            return carry
        lax.fori_loop(0, TPU_T, rbody, 0)

    start_read(0, 0)
    start_read(1, 1)

    def pipe(j, carry):
        for cur in (0, 1):
            k = 2 * j + cur
            pltpu.make_async_copy(
                t2_hbm.at[pl.ds(0, D), pl.ds(0, U_COLS)],
                tins[cur], rsems[cur]).wait()

            @pl.when(j > 0)
            def _():
                pltpu.make_async_copy(
                    touts[cur], out_hbm.at[pl.ds(0, U_WORDS)],
                    wsems[cur]).wait()

            reorder(tins[cur], touts[cur])
            f, tq = unit_uv(k)
            pltpu.async_copy(
                touts[cur],
                out_hbm.at[pl.ds(f * (V * D) + tq * U_WORDS, U_WORDS)],
                wsems[cur])
            start_read(k + 2, cur)
        return carry

    lax.fori_loop(0, KMAX // 2, pipe, 0)
    for cur in (0, 1):
        pltpu.make_async_copy(
            t2_hbm.at[pl.ds(0, D), pl.ds(0, U_COLS)], tins[cur],
            rsems[cur]).wait()
        pltpu.make_async_copy(
            touts[cur], out_hbm.at[pl.ds(0, U_WORDS)], wsems[cur]).wait()

    @pl.when(wid < F_CAT)
    def _():
        f = wid
        pltpu.sync_copy(
            t2_hbm.at[pl.ds(f * D, D), pl.ds(VT_FULL * 128, V_TAIL)], tint)
        lanes8 = lax.iota(jnp.int32, LANES) * D
        for d in range(D):
            for k3 in range(V_TAIL // LANES):
                x = tint[d, pl.ds(k3 * LANES, LANES)]
                plsc.store_scatter(
                    toutt, [(k3 * LANES) * D + d + lanes8], x)
        pltpu.sync_copy(
            toutt, out_hbm.at[pl.ds(f * (V * D) + VT_FULL * 1024,
                                    V_TAIL * D)])


def _sc_transpose(t2):
    return pl.kernel(
        _transpose_body,
        out_type=jax.ShapeDtypeStruct((F_CAT * V * D,), jnp.float32),
        mesh=_MESH,
        scratch_types=[
            pltpu.VMEM((D, U_COLS), jnp.float32),
            pltpu.VMEM((D, U_COLS), jnp.float32),
            pltpu.VMEM((U_WORDS,), jnp.float32),
            pltpu.VMEM((U_WORDS,), jnp.float32),
            pltpu.VMEM((D, V_TAIL), jnp.float32),
            pltpu.VMEM((V_TAIL * D,), jnp.float32),
            pltpu.SemaphoreType.DMA,
            pltpu.SemaphoreType.DMA,
            pltpu.SemaphoreType.DMA,
            pltpu.SemaphoreType.DMA,
        ],
        compiler_params=_SC_TILED,
    )(t2)


def _fv_body(idx_hbm, fv_hbm, fv_v):
    wid = lax.axis_index("s") * NC + lax.axis_index("c")
    base = wid * N_PER_W
    pltpu.sync_copy(idx_hbm.at[pl.ds(base, N_PER_W)], fv_v)

    def flatfv(j, carry):
        sl = pl.ds(j * LANES, LANES)
        pos = base + j * LANES + lax.iota(jnp.int32, LANES)
        fv_v[sl] = fv_v[sl] + lax.rem(pos, F_CAT) * V
        return carry

    lax.fori_loop(0, N_PER_W // LANES, flatfv, 0)
    pltpu.sync_copy(fv_v, fv_hbm.at[pl.ds(base, N_PER_W)])


def _sc_fv(idx_flat):
    return pl.kernel(
        _fv_body,
        out_type=jax.ShapeDtypeStruct((N_LOOK,), jnp.int32),
        mesh=_MESH,
        scratch_types=[pltpu.VMEM((N_PER_W,), jnp.int32)],
        compiler_params=_SC_LIN,
    )(idx_flat)


N_CHUNK = 8
C_ROWS = N_PER_W // N_CHUNK


def _rowgather_body(fv_hbm, table_hbm, out_hbm, ridx_v, rows_v,
                    gsem, w0, w1):
    wid = lax.axis_index("s") * NC + lax.axis_index("c")
    base = wid * N_PER_W
    wsems = (w0, w1)
    wr = [None, None]
    pltpu.sync_copy(fv_hbm.at[pl.ds(base, C_ROWS)], ridx_v.at[0])
    for c in range(N_CHUNK):
        cur = c & 1
        if wr[cur] is not None:
            wr[cur].wait()
        g = pltpu.async_copy(
            table_hbm.at[ridx_v.at[cur]], rows_v.at[cur], gsem)
        if c + 1 < N_CHUNK:
            pltpu.sync_copy(
                fv_hbm.at[pl.ds(base + (c + 1) * C_ROWS, C_ROWS)],
                ridx_v.at[1 - cur])
        g.wait()
        wr[cur] = pltpu.async_copy(
            rows_v.at[cur],
            out_hbm.at[pl.ds(base + c * C_ROWS, C_ROWS)], wsems[cur])
    wr[0].wait()
    wr[1].wait()


def _sc_rowgather(fv, table2):
    return pl.kernel(
        _rowgather_body,
        out_type=jax.ShapeDtypeStruct((N_LOOK, D), jnp.float32),
        mesh=_MESH,
        scratch_types=[
            pltpu.VMEM((2, C_ROWS), jnp.int32),
            pltpu.VMEM((2, C_ROWS, D), jnp.float32),
            pltpu.SemaphoreType.DMA,
            pltpu.SemaphoreType.DMA,
            pltpu.SemaphoreType.DMA,
        ],
        compiler_params=_SC_LIN,
    )(fv, table2)


BLK = 1024


def _mlp_body(cat_ref, nv_ref, nw_ref, nb_ref, w1_ref, b1_ref, w2_ref,
              b2_ref, out_ref):
    catf = cat_ref[...]
    nv = nv_ref[...]
    fi = lax.broadcasted_iota(jnp.int32, (F_NUM, F_NUM * D), 0)
    ji = lax.broadcasted_iota(jnp.int32, (F_NUM, F_NUM * D), 1)
    expand = jnp.where(ji // D == fi, 1.0, 0.0)
    rep = jnp.dot(nv, expand, preferred_element_type=jnp.float32)
    numf = rep * nw_ref[...] + nb_ref[...]
    h = (jnp.dot(catf, w1_ref[0:F_CAT * D, :],
                 preferred_element_type=jnp.float32)
         + jnp.dot(numf, w1_ref[F_CAT * D:, :],
                   preferred_element_type=jnp.float32)
         + b1_ref[...])
    h = jnp.maximum(h, 0.0)
    o = jnp.dot(h, w2_ref[...], preferred_element_type=jnp.float32) + b2_ref[...]
    out_ref[...] = jax.nn.sigmoid(o)


def _mlp(cat_flat, num_values, nw, nb, W1, b1r, W2, b2r):
    grid = (B // BLK,)
    return pl.pallas_call(
        _mlp_body,
        grid=grid,
        in_specs=[
            pl.BlockSpec((BLK, F_CAT * D), lambda i: (i, 0)),
            pl.BlockSpec((BLK, F_NUM), lambda i: (i, 0)),
            pl.BlockSpec((1, F_NUM * D), lambda i: (0, 0)),
            pl.BlockSpec((1, F_NUM * D), lambda i: (0, 0)),
            pl.BlockSpec(((F_CAT + F_NUM) * D, H), lambda i: (0, 0)),
            pl.BlockSpec((1, H), lambda i: (0, 0)),
            pl.BlockSpec((H, 1), lambda i: (0, 0)),
            pl.BlockSpec((1, 1), lambda i: (0, 0)),
        ],
        out_specs=pl.BlockSpec((BLK, 1), lambda i: (i, 0)),
        out_shape=jax.ShapeDtypeStruct((B, 1), jnp.float32),
    )(cat_flat, num_values, nw, nb, W1, b1r, W2, b2r)


def kernel(cat_indices, num_values, emb_tables, num_W, num_b, W1, b1, W2, b2):
    idx_flat = cat_indices.reshape(N_LOOK).astype(jnp.int32)
    t2 = emb_tables.transpose(0, 2, 1).reshape(F_CAT * D, V)
    tv_lin = _sc_transpose(t2)
    fv = _sc_fv(idx_flat)
    table2 = tv_lin.reshape(F_CAT * V, D)
    cat_flat = _sc_rowgather(fv, table2).reshape(B, F_CAT * D)
    nw = num_W.reshape(1, F_NUM * D)
    nb = num_b.reshape(1, F_NUM * D)
    return _mlp(cat_flat, num_values, nw, nb, W1,
                b1.reshape(1, H), W2, b2.reshape(1, 1))


# speedup vs baseline: 1.7820x; 1.1255x over previous
import functools

import jax
import jax.numpy as jnp
from jax import lax
from jax.experimental import pallas as pl
from jax.experimental.pallas import tpu as pltpu
from jax.experimental.pallas import tpu_sc as plsc

B = 16384
F_CAT = 26
F_NUM = 13
V = 100000
D = 8
H = 128

NC, NS = 2, 16
NW = NC * NS
N_LOOK = B * F_CAT
N_PER_W = N_LOOK // NW
LANES = 16

_MESH = plsc.VectorSubcoreMesh(
    core_axis_name="c", subcore_axis_name="s",
    num_cores=NC, num_subcores=NS)
_SC_LIN = pltpu.CompilerParams(
    use_tc_tiling_on_sc=False, needs_layout_passes=False)
_SC_TILED = pltpu.CompilerParams(
    use_tc_tiling_on_sc=True, needs_layout_passes=False)


VT_FULL = V // 128
V_TAIL = V - VT_FULL * 128
TPU_T = 11
U_PER_F = VT_FULL // TPU_T
U_COLS = TPU_T * 128
U_WORDS = U_COLS * D
N_UNIT = F_CAT * U_PER_F
KMAX = 2 * ((N_UNIT + 2 * NW - 1) // (2 * NW))


def _transpose_body(t2_hbm, out_hbm, tin0, tin1, tout0, tout1, tint, toutt,
                    r0, r1, w0, w1):
    wid = lax.axis_index("s") * NC + lax.axis_index("c")
    tins, touts, rsems, wsems = (tin0, tin1), (tout0, tout1), (r0, r1), (w0, w1)

    def unit_uv(k):
        u = jnp.minimum(wid + NW * k, N_UNIT - 1)
        return lax.div(u, U_PER_F), lax.rem(u, U_PER_F)

    def start_read(k, cur):
        f, tq = unit_uv(k)
        pltpu.async_copy(
            t2_hbm.at[pl.ds(f * D, D), pl.ds(tq * U_COLS, U_COLS)],
            tins[cur], rsems[cur])

    def reorder_tile(tin, tout, col0, out0):
        lanes8 = lax.iota(jnp.int32, LANES) * D
        for d in range(D):
            idx = out0 + d + lanes8
            for k3 in range(128 // LANES):
                x = tin[d, pl.ds(col0 + k3 * LANES, LANES)]
                plsc.store_scatter(tout, [idx], x)
                if k3 + 1 < 128 // LANES:
                    idx = idx + LANES * D

    def reorder(tin, tout):
        def rbody(j2, carry):
            reorder_tile(tin, tout, j2 * 128, j2 * 1024)
            return carry
        lax.fori_loop(0, TPU_T, rbody, 0)

    start_read(0, 0)
    start_read(1, 1)

    def pipe(j, carry):
        for cur in (0, 1):
            k = 2 * j + cur
            pltpu.make_async_copy(
                t2_hbm.at[pl.ds(0, D), pl.ds(0, U_COLS)],
                tins[cur], rsems[cur]).wait()

            @pl.when(j > 0)
            def _():
                pltpu.make_async_copy(
                    touts[cur], out_hbm.at[pl.ds(0, U_WORDS)],
                    wsems[cur]).wait()

            reorder(tins[cur], touts[cur])
            f, tq = unit_uv(k)
            pltpu.async_copy(
                touts[cur],
                out_hbm.at[pl.ds(f * (V * D) + tq * U_WORDS, U_WORDS)],
                wsems[cur])
            start_read(k + 2, cur)
        return carry

    lax.fori_loop(0, KMAX // 2, pipe, 0)
    for cur in (0, 1):
        pltpu.make_async_copy(
            t2_hbm.at[pl.ds(0, D), pl.ds(0, U_COLS)], tins[cur],
            rsems[cur]).wait()
        pltpu.make_async_copy(
            touts[cur], out_hbm.at[pl.ds(0, U_WORDS)], wsems[cur]).wait()

    @pl.when(wid < F_CAT)
    def _():
        f = wid
        pltpu.sync_copy(
            t2_hbm.at[pl.ds(f * D, D), pl.ds(VT_FULL * 128, V_TAIL)], tint)
        lanes8 = lax.iota(jnp.int32, LANES) * D
        for d in range(D):
            for k3 in range(V_TAIL // LANES):
                x = tint[d, pl.ds(k3 * LANES, LANES)]
                plsc.store_scatter(
                    toutt, [(k3 * LANES) * D + d + lanes8], x)
        pltpu.sync_copy(
            toutt, out_hbm.at[pl.ds(f * (V * D) + VT_FULL * 1024,
                                    V_TAIL * D)])


def _sc_transpose(t2):
    return pl.kernel(
        _transpose_body,
        out_type=jax.ShapeDtypeStruct((F_CAT * V * D,), jnp.float32),
        mesh=_MESH,
        scratch_types=[
            pltpu.VMEM((D, U_COLS), jnp.float32),
            pltpu.VMEM((D, U_COLS), jnp.float32),
            pltpu.VMEM((U_WORDS,), jnp.float32),
            pltpu.VMEM((U_WORDS,), jnp.float32),
            pltpu.VMEM((D, V_TAIL), jnp.float32),
            pltpu.VMEM((V_TAIL * D,), jnp.float32),
            pltpu.SemaphoreType.DMA,
            pltpu.SemaphoreType.DMA,
            pltpu.SemaphoreType.DMA,
            pltpu.SemaphoreType.DMA,
        ],
        compiler_params=_SC_TILED,
    )(t2)


def _fv_body(idx_hbm, fv_hbm, fv_v):
    wid = lax.axis_index("s") * NC + lax.axis_index("c")
    base = wid * N_PER_W
    pltpu.sync_copy(idx_hbm.at[pl.ds(base, N_PER_W)], fv_v)

    def flatfv(j, carry):
        sl = pl.ds(j * LANES, LANES)
        pos = base + j * LANES + lax.iota(jnp.int32, LANES)
        fv_v[sl] = fv_v[sl] + lax.rem(pos, F_CAT) * V
        return carry

    lax.fori_loop(0, N_PER_W // LANES, flatfv, 0)
    pltpu.sync_copy(fv_v, fv_hbm.at[pl.ds(base, N_PER_W)])


def _sc_fv(idx_flat):
    return pl.kernel(
        _fv_body,
        out_type=jax.ShapeDtypeStruct((N_LOOK,), jnp.int32),
        mesh=_MESH,
        scratch_types=[pltpu.VMEM((N_PER_W,), jnp.int32)],
        compiler_params=_SC_LIN,
    )(idx_flat)


N_CHUNK = 8
C_ROWS = N_PER_W // N_CHUNK


def _rowgather_body(fv_hbm, table_hbm, out_hbm, ridx_v, rows_v,
                    gsem, w0, w1):
    wid = lax.axis_index("s") * NC + lax.axis_index("c")
    base = wid * N_PER_W
    wsems = (w0, w1)
    wr = [None, None]
    pltpu.sync_copy(fv_hbm.at[pl.ds(base, C_ROWS)], ridx_v.at[0])
    for c in range(N_CHUNK):
        cur = c & 1
        if wr[cur] is not None:
            wr[cur].wait()
        g = pltpu.async_copy(
            table_hbm.at[ridx_v.at[cur]], rows_v.at[cur], gsem)
        if c + 1 < N_CHUNK:
            pltpu.sync_copy(
                fv_hbm.at[pl.ds(base + (c + 1) * C_ROWS, C_ROWS)],
                ridx_v.at[1 - cur])
        g.wait()
        wr[cur] = pltpu.async_copy(
            rows_v.at[cur],
            out_hbm.at[pl.ds(base + c * C_ROWS, C_ROWS)], wsems[cur])
    wr[0].wait()
    wr[1].wait()


def _sc_rowgather(fv, table2):
    return pl.kernel(
        _rowgather_body,
        out_type=jax.ShapeDtypeStruct((N_LOOK, D), jnp.float32),
        mesh=_MESH,
        scratch_types=[
            pltpu.VMEM((2, C_ROWS), jnp.int32),
            pltpu.VMEM((2, C_ROWS, D), jnp.float32),
            pltpu.SemaphoreType.DMA,
            pltpu.SemaphoreType.DMA,
            pltpu.SemaphoreType.DMA,
        ],
        compiler_params=_SC_LIN,
    )(fv, table2)


BLK = 1024


def _mlp_body(cat_ref, nv_ref, nw_ref, nb_ref, w1_ref, b1_ref, w2_ref,
              b2_ref, out_ref):
    catf = cat_ref[...]
    nv = nv_ref[...]
    fi = lax.broadcasted_iota(jnp.int32, (F_NUM, F_NUM * D), 0)
    ji = lax.broadcasted_iota(jnp.int32, (F_NUM, F_NUM * D), 1)
    expand = jnp.where(ji // D == fi, 1.0, 0.0)
    rep = jnp.dot(nv, expand, preferred_element_type=jnp.float32)
    numf = rep * nw_ref[...] + nb_ref[...]
    h = (jnp.dot(catf, w1_ref[0:F_CAT * D, :],
                 preferred_element_type=jnp.float32)
         + jnp.dot(numf, w1_ref[F_CAT * D:, :],
                   preferred_element_type=jnp.float32)
         + b1_ref[...])
    h = jnp.maximum(h, 0.0)
    o = jnp.dot(h, w2_ref[...], preferred_element_type=jnp.float32) + b2_ref[...]
    out_ref[...] = jax.nn.sigmoid(o)


def _mlp(cat_flat, num_values, nw, nb, W1, b1r, W2, b2r):
    grid = (B // BLK,)
    return pl.pallas_call(
        _mlp_body,
        grid=grid,
        in_specs=[
            pl.BlockSpec((BLK, F_CAT * D), lambda i: (i, 0)),
            pl.BlockSpec((BLK, F_NUM), lambda i: (i, 0)),
            pl.BlockSpec((1, F_NUM * D), lambda i: (0, 0)),
            pl.BlockSpec((1, F_NUM * D), lambda i: (0, 0)),
            pl.BlockSpec(((F_CAT + F_NUM) * D, H), lambda i: (0, 0)),
            pl.BlockSpec((1, H), lambda i: (0, 0)),
            pl.BlockSpec((H, 1), lambda i: (0, 0)),
            pl.BlockSpec((1, 1), lambda i: (0, 0)),
        ],
        out_specs=pl.BlockSpec((BLK, 1), lambda i: (i, 0)),
        out_shape=jax.ShapeDtypeStruct((B, 1), jnp.float32),
    )(cat_flat, num_values, nw, nb, W1, b1r, W2, b2r)


def kernel(cat_indices, num_values, emb_tables, num_W, num_b, W1, b1, W2, b2):
    idx_flat = cat_indices.reshape(N_LOOK).astype(jnp.int32)
    t2 = emb_tables.transpose(0, 2, 1).reshape(F_CAT * D, V)
    tv_lin = _sc_transpose(t2)
    fv = _sc_fv(idx_flat)
    table2 = tv_lin.reshape(F_CAT * V, D)
    cat_flat = _sc_rowgather(fv, table2).reshape(B, F_CAT * D)
    nw = num_W.reshape(1, F_NUM * D)
    nb = num_b.reshape(1, F_NUM * D)
    return _mlp(cat_flat, num_values, nw, nb, W1,
                b1.reshape(1, H), W2, b2.reshape(1, 1))


# speedup vs baseline: 1.8108x; 1.0161x over previous
import functools

import jax
import jax.numpy as jnp
from jax import lax
from jax.experimental import pallas as pl
from jax.experimental.pallas import tpu as pltpu
from jax.experimental.pallas import tpu_sc as plsc

B = 16384
F_CAT = 26
F_NUM = 13
V = 100000
D = 8
H = 128

NC, NS = 2, 16
NW = NC * NS
N_LOOK = B * F_CAT
N_PER_W = N_LOOK // NW
LANES = 16

_MESH = plsc.VectorSubcoreMesh(
    core_axis_name="c", subcore_axis_name="s",
    num_cores=NC, num_subcores=NS)
_SC_LIN = pltpu.CompilerParams(
    use_tc_tiling_on_sc=False, needs_layout_passes=False)
_SC_TILED = pltpu.CompilerParams(
    use_tc_tiling_on_sc=True, needs_layout_passes=False)


VT_FULL = V // 128
V_TAIL = V - VT_FULL * 128
TPU_T = 11
U_PER_F = VT_FULL // TPU_T
U_COLS = TPU_T * 128
U_WORDS = U_COLS * D
N_UNIT = F_CAT * U_PER_F
KMAX = 2 * ((N_UNIT + 2 * NW - 1) // (2 * NW))


def _transpose_body(t2_hbm, out_hbm, tin0, tin1, tout0, tout1, tint, toutt,
                    r0, r1, w0, w1):
    wid = lax.axis_index("s") * NC + lax.axis_index("c")
    tins, touts, rsems, wsems = (tin0, tin1), (tout0, tout1), (r0, r1), (w0, w1)

    def unit_uv(k):
        u = jnp.minimum(wid + NW * k, N_UNIT - 1)
        return lax.div(u, U_PER_F), lax.rem(u, U_PER_F)

    def start_read(k, cur):
        f, tq = unit_uv(k)
        pltpu.async_copy(
            t2_hbm.at[pl.ds(f * D, D), pl.ds(tq * U_COLS, U_COLS)],
            tins[cur], rsems[cur])

    def reorder(tin, tout):
        lanes8 = lax.iota(jnp.int32, LANES) * D
        init = tuple(d + lanes8 for d in range(D))

        def rbody(j2, idxs):
            col0 = j2 * 128
            nxt = []
            for d in range(D):
                idx = idxs[d]
                for k3 in range(128 // LANES):
                    x = tin[d, pl.ds(col0 + k3 * LANES, LANES)]
                    plsc.store_scatter(tout, [idx], x)
                    idx = idx + LANES * D
                nxt.append(idx)
            return tuple(nxt)

        lax.fori_loop(0, TPU_T, rbody, init)

    start_read(0, 0)
    start_read(1, 1)

    def pipe(j, carry):
        for cur in (0, 1):
            k = 2 * j + cur
            pltpu.make_async_copy(
                t2_hbm.at[pl.ds(0, D), pl.ds(0, U_COLS)],
                tins[cur], rsems[cur]).wait()

            @pl.when(j > 0)
            def _():
                pltpu.make_async_copy(
                    touts[cur], out_hbm.at[pl.ds(0, U_WORDS)],
                    wsems[cur]).wait()

            reorder(tins[cur], touts[cur])
            f, tq = unit_uv(k)
            pltpu.async_copy(
                touts[cur],
                out_hbm.at[pl.ds(f * (V * D) + tq * U_WORDS, U_WORDS)],
                wsems[cur])
            start_read(k + 2, cur)
        return carry

    lax.fori_loop(0, KMAX // 2, pipe, 0)
    for cur in (0, 1):
        pltpu.make_async_copy(
            t2_hbm.at[pl.ds(0, D), pl.ds(0, U_COLS)], tins[cur],
            rsems[cur]).wait()
        pltpu.make_async_copy(
            touts[cur], out_hbm.at[pl.ds(0, U_WORDS)], wsems[cur]).wait()

    @pl.when(wid < F_CAT)
    def _():
        f = wid
        pltpu.sync_copy(
            t2_hbm.at[pl.ds(f * D, D), pl.ds(VT_FULL * 128, V_TAIL)], tint)
        lanes8 = lax.iota(jnp.int32, LANES) * D
        for d in range(D):
            for k3 in range(V_TAIL // LANES):
                x = tint[d, pl.ds(k3 * LANES, LANES)]
                plsc.store_scatter(
                    toutt, [(k3 * LANES) * D + d + lanes8], x)
        pltpu.sync_copy(
            toutt, out_hbm.at[pl.ds(f * (V * D) + VT_FULL * 1024,
                                    V_TAIL * D)])


def _sc_transpose(t2):
    return pl.kernel(
        _transpose_body,
        out_type=jax.ShapeDtypeStruct((F_CAT * V * D,), jnp.float32),
        mesh=_MESH,
        scratch_types=[
            pltpu.VMEM((D, U_COLS), jnp.float32),
            pltpu.VMEM((D, U_COLS), jnp.float32),
            pltpu.VMEM((U_WORDS,), jnp.float32),
            pltpu.VMEM((U_WORDS,), jnp.float32),
            pltpu.VMEM((D, V_TAIL), jnp.float32),
            pltpu.VMEM((V_TAIL * D,), jnp.float32),
            pltpu.SemaphoreType.DMA,
            pltpu.SemaphoreType.DMA,
            pltpu.SemaphoreType.DMA,
            pltpu.SemaphoreType.DMA,
        ],
        compiler_params=_SC_TILED,
    )(t2)


def _fv_body(idx_hbm, fv_hbm, fv_v):
    wid = lax.axis_index("s") * NC + lax.axis_index("c")
    base = wid * N_PER_W
    pltpu.sync_copy(idx_hbm.at[pl.ds(base, N_PER_W)], fv_v)

    def flatfv(j, carry):
        sl = pl.ds(j * LANES, LANES)
        pos = base + j * LANES + lax.iota(jnp.int32, LANES)
        fv_v[sl] = fv_v[sl] + lax.rem(pos, F_CAT) * V
        return carry

    lax.fori_loop(0, N_PER_W // LANES, flatfv, 0)
    pltpu.sync_copy(fv_v, fv_hbm.at[pl.ds(base, N_PER_W)])


def _sc_fv(idx_flat):
    return pl.kernel(
        _fv_body,
        out_type=jax.ShapeDtypeStruct((N_LOOK,), jnp.int32),
        mesh=_MESH,
        scratch_types=[pltpu.VMEM((N_PER_W,), jnp.int32)],
        compiler_params=_SC_LIN,
    )(idx_flat)


N_CHUNK = 8
C_ROWS = N_PER_W // N_CHUNK


def _rowgather_body(fv_hbm, table_hbm, out_hbm, ridx_v, rows_v,
                    gsem, w0, w1):
    wid = lax.axis_index("s") * NC + lax.axis_index("c")
    base = wid * N_PER_W
    wsems = (w0, w1)
    wr = [None, None]
    pltpu.sync_copy(fv_hbm.at[pl.ds(base, C_ROWS)], ridx_v.at[0])
    for c in range(N_CHUNK):
        cur = c & 1
        if wr[cur] is not None:
            wr[cur].wait()
        g = pltpu.async_copy(
            table_hbm.at[ridx_v.at[cur]], rows_v.at[cur], gsem)
        if c + 1 < N_CHUNK:
            pltpu.sync_copy(
                fv_hbm.at[pl.ds(base + (c + 1) * C_ROWS, C_ROWS)],
                ridx_v.at[1 - cur])
        g.wait()
        wr[cur] = pltpu.async_copy(
            rows_v.at[cur],
            out_hbm.at[pl.ds(base + c * C_ROWS, C_ROWS)], wsems[cur])
    wr[0].wait()
    wr[1].wait()


def _sc_rowgather(fv, table2):
    return pl.kernel(
        _rowgather_body,
        out_type=jax.ShapeDtypeStruct((N_LOOK, D), jnp.float32),
        mesh=_MESH,
        scratch_types=[
            pltpu.VMEM((2, C_ROWS), jnp.int32),
            pltpu.VMEM((2, C_ROWS, D), jnp.float32),
            pltpu.SemaphoreType.DMA,
            pltpu.SemaphoreType.DMA,
            pltpu.SemaphoreType.DMA,
        ],
        compiler_params=_SC_LIN,
    )(fv, table2)


BLK = 2048


def _mlp_body(cat_ref, nv_ref, nw_ref, nb_ref, w1_ref, b1_ref, w2_ref,
              b2_ref, out_ref):
    catf = cat_ref[...]
    nv = nv_ref[...]
    fi = lax.broadcasted_iota(jnp.int32, (F_NUM, F_NUM * D), 0)
    ji = lax.broadcasted_iota(jnp.int32, (F_NUM, F_NUM * D), 1)
    expand = jnp.where(ji // D == fi, 1.0, 0.0)
    rep = jnp.dot(nv, expand, preferred_element_type=jnp.float32)
    numf = rep * nw_ref[...] + nb_ref[...]
    h = (jnp.dot(catf, w1_ref[0:F_CAT * D, :],
                 preferred_element_type=jnp.float32)
         + jnp.dot(numf, w1_ref[F_CAT * D:, :],
                   preferred_element_type=jnp.float32)
         + b1_ref[...])
    h = jnp.maximum(h, 0.0)
    o = jnp.dot(h, w2_ref[...], preferred_element_type=jnp.float32) + b2_ref[...]
    out_ref[...] = jax.nn.sigmoid(o)


def _mlp(cat_flat, num_values, nw, nb, W1, b1r, W2, b2r):
    grid = (B // BLK,)
    return pl.pallas_call(
        _mlp_body,
        grid=grid,
        in_specs=[
            pl.BlockSpec((BLK, F_CAT * D), lambda i: (i, 0)),
            pl.BlockSpec((BLK, F_NUM), lambda i: (i, 0)),
            pl.BlockSpec((1, F_NUM * D), lambda i: (0, 0)),
            pl.BlockSpec((1, F_NUM * D), lambda i: (0, 0)),
            pl.BlockSpec(((F_CAT + F_NUM) * D, H), lambda i: (0, 0)),
            pl.BlockSpec((1, H), lambda i: (0, 0)),
            pl.BlockSpec((H, 1), lambda i: (0, 0)),
            pl.BlockSpec((1, 1), lambda i: (0, 0)),
        ],
        out_specs=pl.BlockSpec((BLK, 1), lambda i: (i, 0)),
        out_shape=jax.ShapeDtypeStruct((B, 1), jnp.float32),
    )(cat_flat, num_values, nw, nb, W1, b1r, W2, b2r)


def kernel(cat_indices, num_values, emb_tables, num_W, num_b, W1, b1, W2, b2):
    idx_flat = cat_indices.reshape(N_LOOK).astype(jnp.int32)
    t2 = emb_tables.transpose(0, 2, 1).reshape(F_CAT * D, V)
    tv_lin = _sc_transpose(t2)
    fv = _sc_fv(idx_flat)
    table2 = tv_lin.reshape(F_CAT * V, D)
    cat_flat = _sc_rowgather(fv, table2).reshape(B, F_CAT * D)
    nw = num_W.reshape(1, F_NUM * D)
    nb = num_b.reshape(1, F_NUM * D)
    return _mlp(cat_flat, num_values, nw, nb, W1,
                b1.reshape(1, H), W2, b2.reshape(1, 1))


# speedup vs baseline: 1.8358x; 1.0138x over previous
import functools

import jax
import jax.numpy as jnp
from jax import lax
from jax.experimental import pallas as pl
from jax.experimental.pallas import tpu as pltpu
from jax.experimental.pallas import tpu_sc as plsc

B = 16384
F_CAT = 26
F_NUM = 13
V = 100000
D = 8
H = 128

NC, NS = 2, 16
NW = NC * NS
N_LOOK = B * F_CAT
N_PER_W = N_LOOK // NW
LANES = 16

_MESH = plsc.VectorSubcoreMesh(
    core_axis_name="c", subcore_axis_name="s",
    num_cores=NC, num_subcores=NS)
_SC_LIN = pltpu.CompilerParams(
    use_tc_tiling_on_sc=False, needs_layout_passes=False)
_SC_TILED = pltpu.CompilerParams(
    use_tc_tiling_on_sc=True, needs_layout_passes=False)


VT_FULL = V // 128
V_TAIL = V - VT_FULL * 128
TPU_T = 11
U_PER_F = VT_FULL // TPU_T
U_COLS = TPU_T * 128
U_WORDS = U_COLS * D
N_UNIT = F_CAT * U_PER_F
KMAX = 2 * ((N_UNIT + 2 * NW - 1) // (2 * NW))


def _transpose_body(t2_hbm, out_hbm, tin0, tin1, tout0, tout1, tint, toutt,
                    r0, r1, w0, w1):
    wid = lax.axis_index("s") * NC + lax.axis_index("c")
    tins, touts, rsems, wsems = (tin0, tin1), (tout0, tout1), (r0, r1), (w0, w1)

    def unit_uv(k):
        u = jnp.minimum(wid + NW * k, N_UNIT - 1)
        return lax.div(u, U_PER_F), lax.rem(u, U_PER_F)

    def start_read(k, cur):
        f, tq = unit_uv(k)
        pltpu.async_copy(
            t2_hbm.at[pl.ds(f * D, D), pl.ds(tq * U_COLS, U_COLS)],
            tins[cur], rsems[cur])

    def reorder(tin, tout):
        lanes8 = lax.iota(jnp.int32, LANES) * D
        init = tuple(d + lanes8 for d in range(D))

        def rbody(j2, idxs):
            col0 = j2 * 128
            nxt = []
            for d in range(D):
                idx = idxs[d]
                for k3 in range(128 // LANES):
                    x = tin[d, pl.ds(col0 + k3 * LANES, LANES)]
                    plsc.store_scatter(tout, [idx], x)
                    idx = idx + LANES * D
                nxt.append(idx)
            return tuple(nxt)

        lax.fori_loop(0, TPU_T, rbody, init)

    start_read(0, 0)
    start_read(1, 1)

    def pipe(j, carry):
        for cur in (0, 1):
            k = 2 * j + cur
            pltpu.make_async_copy(
                t2_hbm.at[pl.ds(0, D), pl.ds(0, U_COLS)],
                tins[cur], rsems[cur]).wait()

            @pl.when(j > 0)
            def _():
                pltpu.make_async_copy(
                    touts[cur], out_hbm.at[pl.ds(0, U_WORDS)],
                    wsems[cur]).wait()

            reorder(tins[cur], touts[cur])
            f, tq = unit_uv(k)
            pltpu.async_copy(
                touts[cur],
                out_hbm.at[pl.ds(f * (V * D) + tq * U_WORDS, U_WORDS)],
                wsems[cur])
            start_read(k + 2, cur)
        return carry

    lax.fori_loop(0, KMAX // 2, pipe, 0)
    for cur in (0, 1):
        pltpu.make_async_copy(
            t2_hbm.at[pl.ds(0, D), pl.ds(0, U_COLS)], tins[cur],
            rsems[cur]).wait()
        pltpu.make_async_copy(
            touts[cur], out_hbm.at[pl.ds(0, U_WORDS)], wsems[cur]).wait()

    @pl.when(wid < F_CAT)
    def _():
        f = wid
        pltpu.sync_copy(
            t2_hbm.at[pl.ds(f * D, D), pl.ds(VT_FULL * 128, V_TAIL)], tint)
        lanes8 = lax.iota(jnp.int32, LANES) * D
        for d in range(D):
            for k3 in range(V_TAIL // LANES):
                x = tint[d, pl.ds(k3 * LANES, LANES)]
                plsc.store_scatter(
                    toutt, [(k3 * LANES) * D + d + lanes8], x)
        pltpu.sync_copy(
            toutt, out_hbm.at[pl.ds(f * (V * D) + VT_FULL * 1024,
                                    V_TAIL * D)])


def _sc_transpose(t2):
    return pl.kernel(
        _transpose_body,
        out_type=jax.ShapeDtypeStruct((F_CAT * V * D,), jnp.float32),
        mesh=_MESH,
        scratch_types=[
            pltpu.VMEM((D, U_COLS), jnp.float32),
            pltpu.VMEM((D, U_COLS), jnp.float32),
            pltpu.VMEM((U_WORDS,), jnp.float32),
            pltpu.VMEM((U_WORDS,), jnp.float32),
            pltpu.VMEM((D, V_TAIL), jnp.float32),
            pltpu.VMEM((V_TAIL * D,), jnp.float32),
            pltpu.SemaphoreType.DMA,
            pltpu.SemaphoreType.DMA,
            pltpu.SemaphoreType.DMA,
            pltpu.SemaphoreType.DMA,
        ],
        compiler_params=_SC_TILED,
    )(t2)


def _fv_body(idx_hbm, fv_hbm, fv_v):
    wid = lax.axis_index("s") * NC + lax.axis_index("c")
    base = wid * N_PER_W
    pltpu.sync_copy(idx_hbm.at[pl.ds(base, N_PER_W)], fv_v)

    def flatfv(j, carry):
        sl = pl.ds(j * LANES, LANES)
        pos = base + j * LANES + lax.iota(jnp.int32, LANES)
        fv_v[sl] = fv_v[sl] + lax.rem(pos, F_CAT) * V
        return carry

    lax.fori_loop(0, N_PER_W // LANES, flatfv, 0)
    pltpu.sync_copy(fv_v, fv_hbm.at[pl.ds(base, N_PER_W)])


def _sc_fv(idx_flat):
    return pl.kernel(
        _fv_body,
        out_type=jax.ShapeDtypeStruct((N_LOOK,), jnp.int32),
        mesh=_MESH,
        scratch_types=[pltpu.VMEM((N_PER_W,), jnp.int32)],
        compiler_params=_SC_LIN,
    )(idx_flat)


N_SPLIT = 2
B_HALF = B // N_SPLIT
L_HALF = N_LOOK // N_SPLIT
R_PER_W = L_HALF // NW
N_CHUNK = 4
C_ROWS = R_PER_W // N_CHUNK


def _rowgather_body(half, fv_hbm, table_hbm, out_hbm, ridx_v, rows_v,
                    gsem, w0, w1):
    wid = lax.axis_index("s") * NC + lax.axis_index("c")
    fbase = half * L_HALF + wid * R_PER_W
    obase = wid * R_PER_W
    wsems = (w0, w1)
    wr = [None, None]
    pltpu.sync_copy(fv_hbm.at[pl.ds(fbase, C_ROWS)], ridx_v.at[0])
    for c in range(N_CHUNK):
        cur = c & 1
        if wr[cur] is not None:
            wr[cur].wait()
        g = pltpu.async_copy(
            table_hbm.at[ridx_v.at[cur]], rows_v.at[cur], gsem)
        if c + 1 < N_CHUNK:
            pltpu.sync_copy(
                fv_hbm.at[pl.ds(fbase + (c + 1) * C_ROWS, C_ROWS)],
                ridx_v.at[1 - cur])
        g.wait()
        wr[cur] = pltpu.async_copy(
            rows_v.at[cur],
            out_hbm.at[pl.ds(obase + c * C_ROWS, C_ROWS)], wsems[cur])
    wr[0].wait()
    wr[1].wait()


def _sc_rowgather(fv, table2, half):
    return pl.kernel(
        functools.partial(_rowgather_body, half),
        out_type=jax.ShapeDtypeStruct((L_HALF, D), jnp.float32),
        mesh=_MESH,
        scratch_types=[
            pltpu.VMEM((2, C_ROWS), jnp.int32),
            pltpu.VMEM((2, C_ROWS, D), jnp.float32),
            pltpu.SemaphoreType.DMA,
            pltpu.SemaphoreType.DMA,
            pltpu.SemaphoreType.DMA,
        ],
        compiler_params=_SC_LIN,
        name=f"rowgather_h{half}",
    )(fv, table2)


BLK = 2048


def _mlp_body(cat_ref, nv_ref, nw_ref, nb_ref, w1_ref, b1_ref, w2_ref,
              b2_ref, out_ref):
    catf = cat_ref[...]
    nv = nv_ref[...]
    fi = lax.broadcasted_iota(jnp.int32, (F_NUM, F_NUM * D), 0)
    ji = lax.broadcasted_iota(jnp.int32, (F_NUM, F_NUM * D), 1)
    expand = jnp.where(ji // D == fi, 1.0, 0.0)
    rep = jnp.dot(nv, expand, preferred_element_type=jnp.float32)
    numf = rep * nw_ref[...] + nb_ref[...]
    h = (jnp.dot(catf, w1_ref[0:F_CAT * D, :],
                 preferred_element_type=jnp.float32)
         + jnp.dot(numf, w1_ref[F_CAT * D:, :],
                   preferred_element_type=jnp.float32)
         + b1_ref[...])
    h = jnp.maximum(h, 0.0)
    o = jnp.dot(h, w2_ref[...], preferred_element_type=jnp.float32) + b2_ref[...]
    out_ref[...] = jax.nn.sigmoid(o)


def _mlp(cat_flat, num_values, nw, nb, W1, b1r, W2, b2r):
    nb_rows = cat_flat.shape[0]
    grid = (nb_rows // BLK,)
    return pl.pallas_call(
        _mlp_body,
        grid=grid,
        in_specs=[
            pl.BlockSpec((BLK, F_CAT * D), lambda i: (i, 0)),
            pl.BlockSpec((BLK, F_NUM), lambda i: (i, 0)),
            pl.BlockSpec((1, F_NUM * D), lambda i: (0, 0)),
            pl.BlockSpec((1, F_NUM * D), lambda i: (0, 0)),
            pl.BlockSpec(((F_CAT + F_NUM) * D, H), lambda i: (0, 0)),
            pl.BlockSpec((1, H), lambda i: (0, 0)),
            pl.BlockSpec((H, 1), lambda i: (0, 0)),
            pl.BlockSpec((1, 1), lambda i: (0, 0)),
        ],
        out_specs=pl.BlockSpec((BLK, 1), lambda i: (i, 0)),
        out_shape=jax.ShapeDtypeStruct((nb_rows, 1), jnp.float32),
    )(cat_flat, num_values, nw, nb, W1, b1r, W2, b2r)


def kernel(cat_indices, num_values, emb_tables, num_W, num_b, W1, b1, W2, b2):
    idx_flat = cat_indices.reshape(N_LOOK).astype(jnp.int32)
    t2 = emb_tables.transpose(0, 2, 1).reshape(F_CAT * D, V)
    tv_lin = _sc_transpose(t2)
    fv = _sc_fv(idx_flat)
    table2 = tv_lin.reshape(F_CAT * V, D)
    nw = num_W.reshape(1, F_NUM * D)
    nb = num_b.reshape(1, F_NUM * D)
    b1r, b2r = b1.reshape(1, H), b2.reshape(1, 1)
    outs = []
    for h in range(N_SPLIT):
        cat_h = _sc_rowgather(fv, table2, h).reshape(B_HALF, F_CAT * D)
        nv_h = lax.slice_in_dim(num_values, h * B_HALF, (h + 1) * B_HALF)
        outs.append(_mlp(cat_h, nv_h, nw, nb, W1, b1r, W2, b2r))
    return jnp.concatenate(outs, axis=0)
